# baseline (speedup 1.0000x reference)
import functools

import numpy as np

import jax
import jax.numpy as jnp
from jax import lax
from jax.experimental import pallas as pl
from jax.experimental.pallas import tpu as pltpu
from jax.experimental.pallas import tpu_sc as plsc

N_NODES = 10000
N_EDGES = 160000
D_IN = 256
D_E = 16
D_OUT = 256
D_H = D_OUT // 2

_NC, _NS = 2, 16
_NW = _NC * _NS
_EPW = N_EDGES // _NW
_NSPLIT = 2
_EH = N_EDGES // _NSPLIT
_EPT = _EH // _NS
_CB = 40
_NCHUNK = _EPT // _CB
_NSLOT = 5
_MASK = np.uint32(0xFFFF0000)


def _pack_bf16_pair(lo_f32, hi_f32):
    lo_bits = lax.bitcast_convert_type(
        lo_f32.astype(jnp.bfloat16).astype(jnp.float32), jnp.uint32)
    hi_bits = lax.bitcast_convert_type(
        hi_f32.astype(jnp.bfloat16).astype(jnp.float32), jnp.uint32)
    word = (lo_bits >> 16) | (hi_bits & _MASK)
    return lax.bitcast_convert_type(word, jnp.int32)


def _unpack_bf16_pair(word_i32):
    w = lax.bitcast_convert_type(word_i32, jnp.uint32)
    lo = lax.bitcast_convert_type(w << 16, jnp.float32)
    hi = lax.bitcast_convert_type(w & _MASK, jnp.float32)
    return lo, hi


def _proj_body(x_ref, a_ref, b_ref, hi_ref, hj_ref):
    xb = x_ref[...]
    hi = jnp.dot(xb, a_ref[...], preferred_element_type=jnp.float32)
    hj = jnp.dot(xb, b_ref[...], preferred_element_type=jnp.float32)
    hi_ref[...] = _pack_bf16_pair(hi[:, :D_H], hi[:, D_H:])
    hj_ref[...] = _pack_bf16_pair(hj[:, :D_H], hj[:, D_H:])


def _node_proj(x, A, B):
    blk = 1000
    return pl.pallas_call(
        _proj_body,
        grid=(N_NODES // blk,),
        in_specs=[
            pl.BlockSpec((blk, D_IN), lambda i: (i, 0)),
            pl.BlockSpec((D_IN, D_OUT), lambda i: (0, 0)),
            pl.BlockSpec((D_IN, D_OUT), lambda i: (0, 0)),
        ],
        out_specs=[
            pl.BlockSpec((blk, D_H), lambda i: (i, 0)),
            pl.BlockSpec((blk, D_H), lambda i: (i, 0)),
        ],
        out_shape=[jax.ShapeDtypeStruct((N_NODES, D_H), jnp.int32)] * 2,
    )(x, A, B)


def _fuse_body(gi_ref, gj_ref, ea_ref, c_ref, out_ref):
    ec = jnp.dot(ea_ref[...], c_ref[...], preferred_element_type=jnp.float32)
    gil, gih = _unpack_bf16_pair(gi_ref[...])
    gjl, gjh = _unpack_bf16_pair(gj_ref[...])
    out_ref[:, :D_H] = jnp.maximum(gil + gjl + ec[:, :D_H], 0.0)
    out_ref[:, D_H:] = jnp.maximum(gih + gjh + ec[:, D_H:], 0.0)


_FBLK = 2000


def _edge_fuse_first(gi, gj, ea_half, C):
    return pl.pallas_call(
        _fuse_body,
        grid=(_EH // _FBLK,),
        in_specs=[
            pl.BlockSpec((_FBLK, D_H), lambda i: (i, 0)),
            pl.BlockSpec((_FBLK, D_H), lambda i: (i, 0)),
            pl.BlockSpec((_FBLK, D_E), lambda i: (i, 0)),
            pl.BlockSpec((D_E, D_OUT), lambda i: (0, 0)),
        ],
        out_specs=pl.BlockSpec((_FBLK, D_OUT), lambda i: (i, 0)),
        out_shape=jax.ShapeDtypeStruct((N_EDGES, D_OUT), jnp.float32),
    )(gi, gj, ea_half, C)


def _fuse_body_second(acc_ref, gi_ref, gj_ref, ea_ref, c_ref, out_ref):
    del acc_ref
    _fuse_body(gi_ref, gj_ref, ea_ref, c_ref, out_ref)


def _edge_fuse_second(acc, gi, gj, ea_half, C):
    off = _EH // _FBLK
    return pl.pallas_call(
        _fuse_body_second,
        grid=(_EH // _FBLK,),
        in_specs=[
            pl.BlockSpec(memory_space=pl.ANY),
            pl.BlockSpec((_FBLK, D_H), lambda i: (i, 0)),
            pl.BlockSpec((_FBLK, D_H), lambda i: (i, 0)),
            pl.BlockSpec((_FBLK, D_E), lambda i: (i, 0)),
            pl.BlockSpec((D_E, D_OUT), lambda i: (0, 0)),
        ],
        out_specs=pl.BlockSpec((_FBLK, D_OUT), lambda i: (i + off, 0)),
        out_shape=jax.ShapeDtypeStruct((N_EDGES, D_OUT), jnp.float32),
        input_output_aliases={0: 0},
    )(acc, gi, gj, ea_half, C)


def _sc_body(hi_hbm, hj_hbm, row_hbm, col_hbm, gi_hbm, gj_hbm,
             shared, idx_all, bufs, sems_g, sems_o):
    cid = lax.axis_index("c")
    sid = lax.axis_index("s")

    def pipe(tab_hbm, idx_hbm, out_hbm):
        seg = 624
        pltpu.sync_copy(tab_hbm.at[pl.ds(sid * seg, seg)],
                        shared.at[pl.ds(sid * seg, seg)])

        @pl.when(sid == 0)
        def _():
            tail = N_NODES - seg * _NS
            pltpu.sync_copy(tab_hbm.at[pl.ds(seg * _NS, tail)],
                            shared.at[pl.ds(seg * _NS, tail)])

        pltpu.sync_copy(idx_hbm.at[pl.ds(sid * _EPT, _EPT)], idx_all)
        plsc.subcore_barrier()

        def issue(k, s):
            pltpu.async_copy(shared.at[idx_all.at[pl.ds(k * _CB, _CB)]],
                             bufs[s], sems_g[s])

        def finish(k, s):
            base = (sid * _EPT) + k * _CB
            pltpu.make_async_copy(shared.at[idx_all.at[pl.ds(k * _CB, _CB)]],
                                  bufs[s], sems_g[s]).wait()
            pltpu.async_copy(bufs[s], out_hbm.at[pl.ds(base, _CB)], sems_o[s])

        def wait_out(s):
            pltpu.make_async_copy(bufs[s], out_hbm.at[pl.ds(0, _CB)],
                                  sems_o[s]).wait()

        issue(0, 0)
        issue(1, 1)
        issue(2, 2)

        def group(q, carry):
            k0 = _NSLOT * q
            for s in range(_NSLOT):
                k = k0 + s
                finish(k, s)
                nxt = k + 3
                ns = (s + 3) % _NSLOT

                @pl.when(nxt < _NCHUNK)
                def _():
                    @pl.when(nxt >= _NSLOT)
                    def _():
                        wait_out(ns)

                    issue(nxt, ns)
            return carry

        lax.fori_loop(0, _NCHUNK // _NSLOT, group, 0, unroll=False)
        for s in range(_NSLOT):
            wait_out(s)

    @pl.when(cid == 0)
    def _():
        pipe(hi_hbm, row_hbm, gi_hbm)

    @pl.when(cid == 1)
    def _():
        pipe(hj_hbm, col_hbm, gj_hbm)


def _sc_gather(hi, hj, row3, col3):
    mesh = plsc.VectorSubcoreMesh(core_axis_name="c", subcore_axis_name="s",
                                  num_cores=_NC, num_subcores=_NS)
    f = pl.kernel(
        _sc_body,
        out_type=[jax.ShapeDtypeStruct((_EH, D_H), jnp.int32)] * 2,
        mesh=mesh,
        scratch_types=[
            pltpu.VMEM_SHARED((N_NODES, D_H), jnp.int32),
            pltpu.VMEM((_EPT,), jnp.int32),
            [pltpu.VMEM((_CB, D_H), jnp.int32) for _ in range(_NSLOT)],
            [pltpu.SemaphoreType.DMA for _ in range(_NSLOT)],
            [pltpu.SemaphoreType.DMA for _ in range(_NSLOT)],
        ],
    )
    return f(hi, hj, row3, col3)


def kernel(x, edge_attr, edge_index, edge_type, A, B, C):
    del edge_type
    row = edge_index[0]
    col = edge_index[1]
    hi, hj = _node_proj(x, A, B)
    gi1, gj1 = _sc_gather(hi, hj, row[:_EH], col[:_EH])
    gi2, gj2 = _sc_gather(hi, hj, row[_EH:], col[_EH:])
    out = _edge_fuse_first(gi1, gj1, edge_attr[:_EH], C)
    return _edge_fuse_second(out, gi2, gj2, edge_attr[_EH:], C)


# speedup vs baseline: 1.0495x; 1.0495x over previous
import functools

import numpy as np

import jax
import jax.numpy as jnp
from jax import lax
from jax.experimental import pallas as pl
from jax.experimental.pallas import tpu as pltpu
from jax.experimental.pallas import tpu_sc as plsc

N_NODES = 10000
N_EDGES = 160000
D_IN = 256
D_E = 16
D_OUT = 256
D_H = D_OUT // 2

_NC, _NS = 2, 16
_NW = _NC * _NS
_EPW = N_EDGES // _NW
_NSPLIT = 2
_EH = N_EDGES // _NSPLIT
_EPT = _EH // _NS
_CB = 40
_NCHUNK = _EPT // _CB
_NSLOT = 5
_MASK = np.uint32(0xFFFF0000)


def _pack_bf16_pair(lo_f32, hi_f32):
    lo_bits = lax.bitcast_convert_type(
        lo_f32.astype(jnp.bfloat16).astype(jnp.float32), jnp.uint32)
    hi_bits = lax.bitcast_convert_type(
        hi_f32.astype(jnp.bfloat16).astype(jnp.float32), jnp.uint32)
    word = (lo_bits >> 16) | (hi_bits & _MASK)
    return lax.bitcast_convert_type(word, jnp.int32)


def _unpack_bf16_pair(word_i32):
    w = lax.bitcast_convert_type(word_i32, jnp.uint32)
    lo = lax.bitcast_convert_type(w << 16, jnp.float32)
    hi = lax.bitcast_convert_type(w & _MASK, jnp.float32)
    return lo, hi


def _proj_body(x_ref, a_ref, b_ref, hi_ref, hj_ref):
    xb = x_ref[...]
    hi = jnp.dot(xb, a_ref[...], preferred_element_type=jnp.float32)
    hj = jnp.dot(xb, b_ref[...], preferred_element_type=jnp.float32)
    hi_ref[...] = _pack_bf16_pair(hi[:, :D_H], hi[:, D_H:])
    hj_ref[...] = _pack_bf16_pair(hj[:, :D_H], hj[:, D_H:])


def _node_proj(x, A, B):
    blk = 1000
    return pl.pallas_call(
        _proj_body,
        grid=(N_NODES // blk,),
        in_specs=[
            pl.BlockSpec((blk, D_IN), lambda i: (i, 0)),
            pl.BlockSpec((D_IN, D_OUT), lambda i: (0, 0)),
            pl.BlockSpec((D_IN, D_OUT), lambda i: (0, 0)),
        ],
        out_specs=[
            pl.BlockSpec((blk, D_H), lambda i: (i, 0)),
            pl.BlockSpec((blk, D_H), lambda i: (i, 0)),
        ],
        out_shape=[jax.ShapeDtypeStruct((N_NODES, D_H), jnp.int32)] * 2,
    )(x, A, B)


def _fuse_body(gi_ref, gj_ref, ea_ref, c_ref, out_ref):
    ec = jnp.dot(ea_ref[...], c_ref[...], preferred_element_type=jnp.float32)
    gil, gih = _unpack_bf16_pair(gi_ref[...])
    gjl, gjh = _unpack_bf16_pair(gj_ref[...])
    out_ref[:, :D_H] = jnp.maximum(gil + gjl + ec[:, :D_H], 0.0)
    out_ref[:, D_H:] = jnp.maximum(gih + gjh + ec[:, D_H:], 0.0)


_FBLK = 4000


def _edge_fuse_first(gi, gj, edge_attr, C):
    return pl.pallas_call(
        _fuse_body,
        grid=(_EH // _FBLK,),
        in_specs=[
            pl.BlockSpec((_FBLK, D_H), lambda i: (i, 0)),
            pl.BlockSpec((_FBLK, D_H), lambda i: (i, 0)),
            pl.BlockSpec((_FBLK, D_E), lambda i: (i, 0)),
            pl.BlockSpec((D_E, D_OUT), lambda i: (0, 0)),
        ],
        out_specs=pl.BlockSpec((_FBLK, D_OUT), lambda i: (i, 0)),
        out_shape=jax.ShapeDtypeStruct((N_EDGES, D_OUT), jnp.float32),
    )(gi, gj, edge_attr, C)


def _fuse_body_second(acc_ref, gi_ref, gj_ref, ea_ref, c_ref, out_ref):
    del acc_ref
    _fuse_body(gi_ref, gj_ref, ea_ref, c_ref, out_ref)


def _edge_fuse_second(acc, gi, gj, edge_attr, C):
    off = _EH // _FBLK
    return pl.pallas_call(
        _fuse_body_second,
        grid=(_EH // _FBLK,),
        in_specs=[
            pl.BlockSpec(memory_space=pl.ANY),
            pl.BlockSpec((_FBLK, D_H), lambda i: (i, 0)),
            pl.BlockSpec((_FBLK, D_H), lambda i: (i, 0)),
            pl.BlockSpec((_FBLK, D_E), lambda i: (i + off, 0)),
            pl.BlockSpec((D_E, D_OUT), lambda i: (0, 0)),
        ],
        out_specs=pl.BlockSpec((_FBLK, D_OUT), lambda i: (i + off, 0)),
        out_shape=jax.ShapeDtypeStruct((N_EDGES, D_OUT), jnp.float32),
        input_output_aliases={0: 0},
    )(acc, gi, gj, edge_attr, C)


def _make_sc_body(ebase):
    def _sc_body(hi_hbm, hj_hbm, row_hbm, col_hbm, gi_hbm, gj_hbm,
                 shared, idx_all, bufs, sems_g, sems_o):
        cid = lax.axis_index("c")
        sid = lax.axis_index("s")
        _pipe_impl(ebase, cid, sid, hi_hbm, hj_hbm, row_hbm, col_hbm,
                   gi_hbm, gj_hbm, shared, idx_all, bufs, sems_g, sems_o)
    return _sc_body


def _pipe_impl(ebase, cid, sid, hi_hbm, hj_hbm, row_hbm, col_hbm,
               gi_hbm, gj_hbm, shared, idx_all, bufs, sems_g, sems_o):
    def pipe(tab_hbm, idx_hbm, out_hbm):
        seg = 624
        pltpu.sync_copy(tab_hbm.at[pl.ds(sid * seg, seg)],
                        shared.at[pl.ds(sid * seg, seg)])

        @pl.when(sid == 0)
        def _():
            tail = N_NODES - seg * _NS
            pltpu.sync_copy(tab_hbm.at[pl.ds(seg * _NS, tail)],
                            shared.at[pl.ds(seg * _NS, tail)])

        pltpu.sync_copy(idx_hbm.at[pl.ds(ebase + sid * _EPT, _EPT)], idx_all)
        plsc.subcore_barrier()

        def issue(k, s):
            pltpu.async_copy(shared.at[idx_all.at[pl.ds(k * _CB, _CB)]],
                             bufs[s], sems_g[s])

        def finish(k, s):
            base = (sid * _EPT) + k * _CB
            pltpu.make_async_copy(shared.at[idx_all.at[pl.ds(k * _CB, _CB)]],
                                  bufs[s], sems_g[s]).wait()
            pltpu.async_copy(bufs[s], out_hbm.at[pl.ds(base, _CB)], sems_o[s])

        def wait_out(s):
            pltpu.make_async_copy(bufs[s], out_hbm.at[pl.ds(0, _CB)],
                                  sems_o[s]).wait()

        issue(0, 0)
        issue(1, 1)
        issue(2, 2)

        def group(q, carry):
            k0 = _NSLOT * q
            for s in range(_NSLOT):
                k = k0 + s
                finish(k, s)
                nxt = k + 3
                ns = (s + 3) % _NSLOT

                @pl.when(nxt < _NCHUNK)
                def _():
                    @pl.when(nxt >= _NSLOT)
                    def _():
                        wait_out(ns)

                    issue(nxt, ns)
            return carry

        lax.fori_loop(0, _NCHUNK // _NSLOT, group, 0, unroll=False)
        for s in range(_NSLOT):
            wait_out(s)

    @pl.when(cid == 0)
    def _():
        pipe(hi_hbm, row_hbm, gi_hbm)

    @pl.when(cid == 1)
    def _():
        pipe(hj_hbm, col_hbm, gj_hbm)


def _sc_gather(hi, hj, row, col, ebase):
    mesh = plsc.VectorSubcoreMesh(core_axis_name="c", subcore_axis_name="s",
                                  num_cores=_NC, num_subcores=_NS)
    f = pl.kernel(
        _make_sc_body(ebase),
        out_type=[jax.ShapeDtypeStruct((_EH, D_H), jnp.int32)] * 2,
        mesh=mesh,
        scratch_types=[
            pltpu.VMEM_SHARED((N_NODES, D_H), jnp.int32),
            pltpu.VMEM((_EPT,), jnp.int32),
            [pltpu.VMEM((_CB, D_H), jnp.int32) for _ in range(_NSLOT)],
            [pltpu.SemaphoreType.DMA for _ in range(_NSLOT)],
            [pltpu.SemaphoreType.DMA for _ in range(_NSLOT)],
        ],
    )
    return f(hi, hj, row, col)


def kernel(x, edge_attr, edge_index, edge_type, A, B, C):
    del edge_type
    row = edge_index[0]
    col = edge_index[1]
    hi, hj = _node_proj(x, A, B)
    gi1, gj1 = _sc_gather(hi, hj, row, col, 0)
    gi2, gj2 = _sc_gather(hi, hj, row, col, _EH)
    out = _edge_fuse_first(gi1, gj1, edge_attr, C)
    return _edge_fuse_second(out, gi2, gj2, edge_attr, C)


# speedup vs baseline: 1.0549x; 1.0052x over previous
import functools

import numpy as np

import jax
import jax.numpy as jnp
from jax import lax
from jax.experimental import pallas as pl
from jax.experimental.pallas import tpu as pltpu
from jax.experimental.pallas import tpu_sc as plsc

N_NODES = 10000
N_EDGES = 160000
D_IN = 256
D_E = 16
D_OUT = 256
D_H = D_OUT // 2

_NC, _NS = 2, 16
_NW = _NC * _NS
_EPW = N_EDGES // _NW
_NSPLIT = 2
_EH = N_EDGES // _NSPLIT
_EPT = _EH // _NS
_CB = 40
_NCHUNK = _EPT // _CB
_NSLOT = 5
_MASK = np.uint32(0xFFFF0000)


def _pack_bf16_pair(lo_f32, hi_f32):
    lo_bits = lax.bitcast_convert_type(
        lo_f32.astype(jnp.bfloat16).astype(jnp.float32), jnp.uint32)
    hi_bits = lax.bitcast_convert_type(
        hi_f32.astype(jnp.bfloat16).astype(jnp.float32), jnp.uint32)
    word = (lo_bits >> 16) | (hi_bits & _MASK)
    return lax.bitcast_convert_type(word, jnp.int32)


def _unpack_bf16_pair(word_i32):
    w = lax.bitcast_convert_type(word_i32, jnp.uint32)
    lo = lax.bitcast_convert_type(w << 16, jnp.float32)
    hi = lax.bitcast_convert_type(w & _MASK, jnp.float32)
    return lo, hi


def _proj_body(x_ref, a_ref, b_ref, hi_ref, hj_ref):
    xb = x_ref[...]
    hi = jnp.dot(xb, a_ref[...], preferred_element_type=jnp.float32)
    hj = jnp.dot(xb, b_ref[...], preferred_element_type=jnp.float32)
    hi_ref[...] = _pack_bf16_pair(hi[:, :D_H], hi[:, D_H:])
    hj_ref[...] = _pack_bf16_pair(hj[:, :D_H], hj[:, D_H:])


def _node_proj(x, A, B):
    blk = 1000
    return pl.pallas_call(
        _proj_body,
        grid=(N_NODES // blk,),
        in_specs=[
            pl.BlockSpec((blk, D_IN), lambda i: (i, 0)),
            pl.BlockSpec((D_IN, D_OUT), lambda i: (0, 0)),
            pl.BlockSpec((D_IN, D_OUT), lambda i: (0, 0)),
        ],
        out_specs=[
            pl.BlockSpec((blk, D_H), lambda i: (i, 0)),
            pl.BlockSpec((blk, D_H), lambda i: (i, 0)),
        ],
        out_shape=[jax.ShapeDtypeStruct((N_NODES, D_H), jnp.int32)] * 2,
    )(x, A, B)


def _fuse_body(gi_ref, gj_ref, ea_ref, c_ref, out_ref):
    ec = jnp.dot(ea_ref[...], c_ref[...], preferred_element_type=jnp.float32)
    gil, gih = _unpack_bf16_pair(gi_ref[...])
    gjl, gjh = _unpack_bf16_pair(gj_ref[...])
    out_ref[:, :D_H] = jnp.maximum(gil + gjl + ec[:, :D_H], 0.0)
    out_ref[:, D_H:] = jnp.maximum(gih + gjh + ec[:, D_H:], 0.0)


_FBLK = 8000


def _edge_fuse_first(gi, gj, edge_attr, C):
    return pl.pallas_call(
        _fuse_body,
        grid=(_EH // _FBLK,),
        in_specs=[
            pl.BlockSpec((_FBLK, D_H), lambda i: (i, 0)),
            pl.BlockSpec((_FBLK, D_H), lambda i: (i, 0)),
            pl.BlockSpec((_FBLK, D_E), lambda i: (i, 0)),
            pl.BlockSpec((D_E, D_OUT), lambda i: (0, 0)),
        ],
        out_specs=pl.BlockSpec((_FBLK, D_OUT), lambda i: (i, 0)),
        out_shape=jax.ShapeDtypeStruct((N_EDGES, D_OUT), jnp.float32),
    )(gi, gj, edge_attr, C)


def _fuse_body_second(acc_ref, gi_ref, gj_ref, ea_ref, c_ref, out_ref):
    del acc_ref
    _fuse_body(gi_ref, gj_ref, ea_ref, c_ref, out_ref)


def _edge_fuse_second(acc, gi, gj, edge_attr, C):
    off = _EH // _FBLK
    return pl.pallas_call(
        _fuse_body_second,
        grid=(_EH // _FBLK,),
        in_specs=[
            pl.BlockSpec(memory_space=pl.ANY),
            pl.BlockSpec((_FBLK, D_H), lambda i: (i, 0)),
            pl.BlockSpec((_FBLK, D_H), lambda i: (i, 0)),
            pl.BlockSpec((_FBLK, D_E), lambda i: (i + off, 0)),
            pl.BlockSpec((D_E, D_OUT), lambda i: (0, 0)),
        ],
        out_specs=pl.BlockSpec((_FBLK, D_OUT), lambda i: (i + off, 0)),
        out_shape=jax.ShapeDtypeStruct((N_EDGES, D_OUT), jnp.float32),
        input_output_aliases={0: 0},
    )(acc, gi, gj, edge_attr, C)


def _make_sc_body(ebase):
    def _sc_body(hi_hbm, hj_hbm, row_hbm, col_hbm, gi_hbm, gj_hbm,
                 shared, idx_all, bufs, sems_g, sems_o):
        cid = lax.axis_index("c")
        sid = lax.axis_index("s")
        _pipe_impl(ebase, cid, sid, hi_hbm, hj_hbm, row_hbm, col_hbm,
                   gi_hbm, gj_hbm, shared, idx_all, bufs, sems_g, sems_o)
    return _sc_body


def _pipe_impl(ebase, cid, sid, hi_hbm, hj_hbm, row_hbm, col_hbm,
               gi_hbm, gj_hbm, shared, idx_all, bufs, sems_g, sems_o):
    def pipe(tab_hbm, idx_hbm, out_hbm):
        seg = 624
        pltpu.sync_copy(tab_hbm.at[pl.ds(sid * seg, seg)],
                        shared.at[pl.ds(sid * seg, seg)])

        @pl.when(sid == 0)
        def _():
            tail = N_NODES - seg * _NS
            pltpu.sync_copy(tab_hbm.at[pl.ds(seg * _NS, tail)],
                            shared.at[pl.ds(seg * _NS, tail)])

        pltpu.sync_copy(idx_hbm.at[pl.ds(ebase + sid * _EPT, _EPT)], idx_all)
        plsc.subcore_barrier()

        def issue(k, s):
            pltpu.async_copy(shared.at[idx_all.at[pl.ds(k * _CB, _CB)]],
                             bufs[s], sems_g[s])

        def finish(k, s):
            base = (sid * _EPT) + k * _CB
            pltpu.make_async_copy(shared.at[idx_all.at[pl.ds(k * _CB, _CB)]],
                                  bufs[s], sems_g[s]).wait()
            pltpu.async_copy(bufs[s], out_hbm.at[pl.ds(base, _CB)], sems_o[s])

        def wait_out(s):
            pltpu.make_async_copy(bufs[s], out_hbm.at[pl.ds(0, _CB)],
                                  sems_o[s]).wait()

        issue(0, 0)
        issue(1, 1)
        issue(2, 2)

        def group(q, carry):
            k0 = _NSLOT * q
            for s in range(_NSLOT):
                k = k0 + s
                finish(k, s)
                nxt = k + 3
                ns = (s + 3) % _NSLOT

                @pl.when(nxt < _NCHUNK)
                def _():
                    @pl.when(nxt >= _NSLOT)
                    def _():
                        wait_out(ns)

                    issue(nxt, ns)
            return carry

        lax.fori_loop(0, _NCHUNK // _NSLOT, group, 0, unroll=False)
        for s in range(_NSLOT):
            wait_out(s)

    @pl.when(cid == 0)
    def _():
        pipe(hi_hbm, row_hbm, gi_hbm)

    @pl.when(cid == 1)
    def _():
        pipe(hj_hbm, col_hbm, gj_hbm)


def _sc_gather(hi, hj, row, col, ebase):
    mesh = plsc.VectorSubcoreMesh(core_axis_name="c", subcore_axis_name="s",
                                  num_cores=_NC, num_subcores=_NS)
    f = pl.kernel(
        _make_sc_body(ebase),
        out_type=[jax.ShapeDtypeStruct((_EH, D_H), jnp.int32)] * 2,
        mesh=mesh,
        scratch_types=[
            pltpu.VMEM_SHARED((N_NODES, D_H), jnp.int32),
            pltpu.VMEM((_EPT,), jnp.int32),
            [pltpu.VMEM((_CB, D_H), jnp.int32) for _ in range(_NSLOT)],
            [pltpu.SemaphoreType.DMA for _ in range(_NSLOT)],
            [pltpu.SemaphoreType.DMA for _ in range(_NSLOT)],
        ],
    )
    return f(hi, hj, row, col)


def kernel(x, edge_attr, edge_index, edge_type, A, B, C):
    del edge_type
    row = edge_index[0]
    col = edge_index[1]
    hi, hj = _node_proj(x, A, B)
    gi1, gj1 = _sc_gather(hi, hj, row, col, 0)
    gi2, gj2 = _sc_gather(hi, hj, row, col, _EH)
    out = _edge_fuse_first(gi1, gj1, edge_attr, C)
    return _edge_fuse_second(out, gi2, gj2, edge_attr, C)
